# baseline (speedup 1.0000x reference)
import functools
import jax
import jax.numpy as jnp
from jax import lax
from jax.experimental import pallas as pl
from jax.experimental.pallas import tpu as pltpu
from jax.experimental.pallas import tpu_sc as plsc

NUM_CORES = 2
NUM_SUBCORES = 16
NUM_WORKERS = NUM_CORES * NUM_SUBCORES
LANES = 16
CHUNK_ROWS = 32


def kernel(x, pos_emb):
    batch, seq_len, d_model = x.shape
    xf = x.reshape(-1)
    pef = pos_emb[:seq_len].reshape(-1)
    total_rows = batch * seq_len
    rows_per_w = total_rows // NUM_WORKERS
    assert total_rows % NUM_WORKERS == 0 and seq_len % rows_per_w == 0
    chunk_words = CHUNK_ROWS * d_model
    n_chunks = (rows_per_w * d_model) // chunk_words

    mesh = plsc.VectorSubcoreMesh(core_axis_name="c", subcore_axis_name="s")

    @functools.partial(
        pl.kernel,
        out_type=jax.ShapeDtypeStruct((total_rows * d_model,), jnp.float32),
        mesh=mesh,
        scratch_types=[
            pltpu.VMEM((chunk_words,), jnp.float32),
            pltpu.VMEM((chunk_words,), jnp.float32),
        ],
    )
    def sc_add(x_hbm, pe_hbm, o_hbm, bufx, bufpe):
        wid = lax.axis_index("s") * NUM_CORES + lax.axis_index("c")
        base = wid * (rows_per_w * d_model)
        pe_base = lax.rem(wid * rows_per_w, seq_len) * d_model

        @pl.loop(0, n_chunks)
        def _chunk(i):
            off = base + i * chunk_words
            pe_off = pe_base + i * chunk_words
            pltpu.sync_copy(x_hbm.at[pl.ds(off, chunk_words)], bufx)
            pltpu.sync_copy(pe_hbm.at[pl.ds(pe_off, chunk_words)], bufpe)

            @pl.loop(0, chunk_words // LANES, unroll=8)
            def _vec(j):
                sl = pl.ds(j * LANES, LANES)
                bufx[sl] = bufx[sl] + bufpe[sl]

            pltpu.sync_copy(bufx, o_hbm.at[pl.ds(off, chunk_words)])

    return sc_add(xf, pef).reshape(x.shape)


# speedup vs baseline: 1.7506x; 1.7506x over previous
import functools
import jax
import jax.numpy as jnp
from jax import lax
from jax.experimental import pallas as pl
from jax.experimental.pallas import tpu as pltpu
from jax.experimental.pallas import tpu_sc as plsc

NUM_CORES = 2
NUM_SUBCORES = 16
NUM_WORKERS = NUM_CORES * NUM_SUBCORES
LANES = 16
CHUNK_ROWS = 8
NBUF = 4


def kernel(x, pos_emb):
    batch, seq_len, d_model = x.shape
    xf = x.reshape(-1)
    pef = pos_emb[:seq_len].reshape(-1)
    total_rows = batch * seq_len
    rows_per_w = total_rows // NUM_WORKERS
    assert total_rows % NUM_WORKERS == 0 and seq_len % rows_per_w == 0
    chunk_words = CHUNK_ROWS * d_model
    n_chunks = (rows_per_w * d_model) // chunk_words
    assert n_chunks % NBUF == 0 and n_chunks >= 2 * NBUF

    mesh = plsc.VectorSubcoreMesh(core_axis_name="c", subcore_axis_name="s")

    @functools.partial(
        pl.kernel,
        out_type=jax.ShapeDtypeStruct((total_rows * d_model,), jnp.float32),
        mesh=mesh,
        scratch_types=(
            [pltpu.VMEM((chunk_words,), jnp.float32) for _ in range(NBUF)]
            + [pltpu.VMEM((chunk_words,), jnp.float32) for _ in range(NBUF)]
            + [pltpu.SemaphoreType.DMA for _ in range(3 * NBUF)]
        ),
    )
    def sc_add(x_hbm, pe_hbm, o_hbm, *scratch):
        bufx = scratch[:NBUF]
        bufpe = scratch[NBUF : 2 * NBUF]
        ldx_sem = scratch[2 * NBUF : 3 * NBUF]
        ldpe_sem = scratch[3 * NBUF : 4 * NBUF]
        st_sem = scratch[4 * NBUF : 5 * NBUF]

        wid = lax.axis_index("s") * NUM_CORES + lax.axis_index("c")
        base = wid * (rows_per_w * d_model)
        pe_base = lax.rem(wid * rows_per_w, seq_len) * d_model

        def start_load(ci, b):
            off = base + ci * chunk_words
            pltpu.async_copy(
                x_hbm.at[pl.ds(off, chunk_words)], bufx[b], ldx_sem[b]
            )
            pltpu.async_copy(
                pe_hbm.at[pl.ds(pe_base + ci * chunk_words, chunk_words)],
                bufpe[b],
                ldpe_sem[b],
            )

        def wait_load(b):
            pltpu.make_async_copy(x_hbm.at[pl.ds(0, chunk_words)], bufx[b], ldx_sem[b]).wait()
            pltpu.make_async_copy(pe_hbm.at[pl.ds(0, chunk_words)], bufpe[b], ldpe_sem[b]).wait()

        def start_store(ci, b):
            off = base + ci * chunk_words
            pltpu.async_copy(bufx[b], o_hbm.at[pl.ds(off, chunk_words)], st_sem[b])

        def wait_store(b):
            pltpu.make_async_copy(bufx[b], o_hbm.at[pl.ds(0, chunk_words)], st_sem[b]).wait()

        start_load(0, 0)
        start_load(1, 1)

        @pl.loop(0, n_chunks, step=NBUF)
        def _g(g):
            for b in range(NBUF):
                ci = g + b
                nxt = ci + 2
                nxt_b = (b + 2) % NBUF

                @pl.when(nxt < n_chunks)
                def _():
                    @pl.when(ci >= NBUF - 2)
                    def _():
                        wait_store(nxt_b)

                    start_load(nxt, nxt_b)

                wait_load(b)

                @pl.loop(0, chunk_words // LANES, unroll=8)
                def _vec(j):
                    sl = pl.ds(j * LANES, LANES)
                    plsc.addupdate(bufx[b].at[sl], bufpe[b][sl])

                start_store(ci, b)

        for b in range(NBUF):
            wait_store(b)

    return sc_add(xf, pef).reshape(x.shape)


# speedup vs baseline: 1.7566x; 1.0034x over previous
import functools
import jax
import jax.numpy as jnp
from jax import lax
from jax.experimental import pallas as pl
from jax.experimental.pallas import tpu as pltpu
from jax.experimental.pallas import tpu_sc as plsc

NUM_CORES = 2
NUM_SUBCORES = 16
NUM_WORKERS = NUM_CORES * NUM_SUBCORES
LANES = 16
CHUNK_ROWS = 8
NBUF = 4


def kernel(x, pos_emb):
    batch, seq_len, d_model = x.shape
    xf = x.reshape(-1)
    pef = pos_emb[:seq_len].reshape(-1)
    total_rows = batch * seq_len
    rows_per_w = total_rows // NUM_WORKERS
    assert total_rows % NUM_WORKERS == 0 and seq_len % rows_per_w == 0
    chunk_words = CHUNK_ROWS * d_model
    n_chunks = (rows_per_w * d_model) // chunk_words
    assert n_chunks % NBUF == 0 and n_chunks >= 2 * NBUF

    mesh = plsc.VectorSubcoreMesh(core_axis_name="c", subcore_axis_name="s")

    @functools.partial(
        pl.kernel,
        out_type=jax.ShapeDtypeStruct((total_rows * d_model,), jnp.float32),
        mesh=mesh,
        scratch_types=(
            [pltpu.VMEM((chunk_words,), jnp.float32) for _ in range(NBUF)]
            + [pltpu.VMEM((chunk_words,), jnp.float32) for _ in range(NBUF)]
            + [pltpu.SemaphoreType.DMA for _ in range(3 * NBUF)]
        ),
    )
    def sc_add(x_hbm, pe_hbm, o_hbm, *scratch):
        bufx = scratch[:NBUF]
        bufpe = scratch[NBUF : 2 * NBUF]
        ldx_sem = scratch[2 * NBUF : 3 * NBUF]
        ldpe_sem = scratch[3 * NBUF : 4 * NBUF]
        st_sem = scratch[4 * NBUF : 5 * NBUF]

        wid = lax.axis_index("s") * NUM_CORES + lax.axis_index("c")
        base = wid * (rows_per_w * d_model)
        pe_base = lax.rem(wid * rows_per_w, seq_len) * d_model

        def start_load(ci, b):
            off = base + ci * chunk_words
            pltpu.async_copy(
                x_hbm.at[pl.ds(off, chunk_words)], bufx[b], ldx_sem[b]
            )
            pltpu.async_copy(
                pe_hbm.at[pl.ds(pe_base + ci * chunk_words, chunk_words)],
                bufpe[b],
                ldpe_sem[b],
            )

        def wait_load(b):
            pltpu.make_async_copy(x_hbm.at[pl.ds(0, chunk_words)], bufx[b], ldx_sem[b]).wait()
            pltpu.make_async_copy(pe_hbm.at[pl.ds(0, chunk_words)], bufpe[b], ldpe_sem[b]).wait()

        def start_store(ci, b):
            off = base + ci * chunk_words
            pltpu.async_copy(bufx[b], o_hbm.at[pl.ds(off, chunk_words)], st_sem[b])

        def wait_store(b):
            pltpu.make_async_copy(bufx[b], o_hbm.at[pl.ds(0, chunk_words)], st_sem[b]).wait()

        start_load(0, 0)
        start_load(1, 1)

        @pl.loop(0, n_chunks, step=NBUF)
        def _g(g):
            for b in range(NBUF):
                ci = g + b
                nxt = ci + 2
                nxt_b = (b + 2) % NBUF

                @pl.when(nxt < n_chunks)
                def _():
                    @pl.when(ci >= NBUF - 2)
                    def _():
                        wait_store(nxt_b)

                    start_load(nxt, nxt_b)

                wait_load(b)

                @plsc.parallel_loop(0, chunk_words, step=LANES, unroll=8)
                def _vec(j):
                    sl = pl.ds(j, LANES)
                    plsc.addupdate(bufx[b].at[sl], bufpe[b][sl])

                start_store(ci, b)

        for b in range(NBUF):
            wait_store(b)

    return sc_add(xf, pef).reshape(x.shape)


# speedup vs baseline: 4.3670x; 2.4861x over previous
import functools
import jax
import jax.numpy as jnp
from jax import lax
from jax.experimental import pallas as pl
from jax.experimental.pallas import tpu as pltpu
from jax.experimental.pallas import tpu_sc as plsc

NUM_CORES = 2
NUM_SUBCORES = 16
NUM_WORKERS = NUM_CORES * NUM_SUBCORES
LANES = 16
CHUNK_ROWS = 8
NBUF = 4


def kernel(x, pos_emb):
    batch, seq_len, d_model = x.shape
    xf = x.reshape(batch * seq_len, d_model)
    pef = pos_emb[:seq_len]
    total_rows = batch * seq_len
    rows_per_w = total_rows // NUM_WORKERS
    assert total_rows % NUM_WORKERS == 0 and seq_len % rows_per_w == 0
    n_chunks = rows_per_w // CHUNK_ROWS
    assert n_chunks % NBUF == 0 and n_chunks >= 2 * NBUF

    mesh = plsc.VectorSubcoreMesh(core_axis_name="c", subcore_axis_name="s")

    @functools.partial(
        pl.kernel,
        out_type=jax.ShapeDtypeStruct((total_rows, d_model), jnp.float32),
        mesh=mesh,
        scratch_types=(
            [pltpu.VMEM((CHUNK_ROWS, d_model), jnp.float32) for _ in range(NBUF)]
            + [pltpu.VMEM((CHUNK_ROWS, d_model), jnp.float32) for _ in range(NBUF)]
            + [pltpu.SemaphoreType.DMA for _ in range(3 * NBUF)]
        ),
    )
    def sc_add(x_hbm, pe_hbm, o_hbm, *scratch):
        bufx = scratch[:NBUF]
        bufpe = scratch[NBUF : 2 * NBUF]
        ldx_sem = scratch[2 * NBUF : 3 * NBUF]
        ldpe_sem = scratch[3 * NBUF : 4 * NBUF]
        st_sem = scratch[4 * NBUF : 5 * NBUF]

        wid = lax.axis_index("s") * NUM_CORES + lax.axis_index("c")
        row0 = wid * rows_per_w
        pe_row0 = lax.rem(row0, seq_len)

        def start_load(ci, b):
            row = row0 + ci * CHUNK_ROWS
            pltpu.async_copy(
                x_hbm.at[pl.ds(row, CHUNK_ROWS)], bufx[b], ldx_sem[b]
            )
            pltpu.async_copy(
                pe_hbm.at[pl.ds(pe_row0 + ci * CHUNK_ROWS, CHUNK_ROWS)],
                bufpe[b],
                ldpe_sem[b],
            )

        def wait_load(b):
            pltpu.make_async_copy(x_hbm.at[pl.ds(0, CHUNK_ROWS)], bufx[b], ldx_sem[b]).wait()
            pltpu.make_async_copy(pe_hbm.at[pl.ds(0, CHUNK_ROWS)], bufpe[b], ldpe_sem[b]).wait()

        def start_store(ci, b):
            row = row0 + ci * CHUNK_ROWS
            pltpu.async_copy(bufx[b], o_hbm.at[pl.ds(row, CHUNK_ROWS)], st_sem[b])

        def wait_store(b):
            pltpu.make_async_copy(bufx[b], o_hbm.at[pl.ds(0, CHUNK_ROWS)], st_sem[b]).wait()

        start_load(0, 0)
        start_load(1, 1)

        @pl.loop(0, n_chunks, step=NBUF)
        def _g(g):
            for b in range(NBUF):
                ci = g + b
                nxt = ci + 2
                nxt_b = (b + 2) % NBUF

                @pl.when(nxt < n_chunks)
                def _():
                    @pl.when(ci >= NBUF - 2)
                    def _():
                        wait_store(nxt_b)

                    start_load(nxt, nxt_b)

                wait_load(b)

                @plsc.parallel_loop(0, d_model, step=LANES, unroll=2)
                def _vec(c):
                    sl = pl.ds(c, LANES)
                    for r in range(CHUNK_ROWS):
                        plsc.addupdate(bufx[b].at[r].at[sl], bufpe[b].at[r][sl])

                start_store(ci, b)

        for b in range(NBUF):
            wait_store(b)

    return sc_add(xf, pef).reshape(x.shape)
